# baseline (speedup 1.0000x reference)
import jax
import jax.numpy as jnp
from jax import lax
from jax.experimental import pallas as pl
from jax.experimental.pallas import tpu as pltpu
from jax.experimental.pallas import tpu_sc as plsc

N_FIELDS = 26
VOCAB = 100000
EMB_DIM = 32
BATCH = 16384
NC, NS = 2, 16
NW = NC * NS
B_PER_W = BATCH // NW


def _body(idx_hbm, *rest):
    tables = rest[:N_FIELDS]
    out_hbm = rest[N_FIELDS]
    idx_v = rest[N_FIELDS + 1]
    rows_v = rest[N_FIELDS + 2]
    sem = rest[N_FIELDS + 3]

    wid = lax.axis_index("s") * NC + lax.axis_index("c")
    base = wid * B_PER_W
    for f in range(N_FIELDS):
        pltpu.sync_copy(idx_hbm.at[f, pl.ds(base, B_PER_W)], idx_v)
        pltpu.async_copy(tables[f].at[idx_v], rows_v, sem).wait()
        pltpu.sync_copy(
            rows_v,
            out_hbm.at[pl.ds(base, B_PER_W), pl.ds(f * EMB_DIM, EMB_DIM)],
        )


def kernel(idx_0, idx_1, idx_2, idx_3, idx_4, idx_5, idx_6, idx_7, idx_8, idx_9, idx_10, idx_11, idx_12, idx_13, idx_14, idx_15, idx_16, idx_17, idx_18, idx_19, idx_20, idx_21, idx_22, idx_23, idx_24, idx_25, table_0, table_1, table_2, table_3, table_4, table_5, table_6, table_7, table_8, table_9, table_10, table_11, table_12, table_13, table_14, table_15, table_16, table_17, table_18, table_19, table_20, table_21, table_22, table_23, table_24, table_25):
    idxs = [
        idx_0, idx_1, idx_2, idx_3, idx_4, idx_5, idx_6, idx_7, idx_8, idx_9,
        idx_10, idx_11, idx_12, idx_13, idx_14, idx_15, idx_16, idx_17,
        idx_18, idx_19, idx_20, idx_21, idx_22, idx_23, idx_24, idx_25,
    ]
    tables = [
        table_0, table_1, table_2, table_3, table_4, table_5, table_6,
        table_7, table_8, table_9, table_10, table_11, table_12, table_13,
        table_14, table_15, table_16, table_17, table_18, table_19, table_20,
        table_21, table_22, table_23, table_24, table_25,
    ]
    idx_all = jnp.stack(idxs).astype(jnp.int32)

    k = pl.kernel(
        _body,
        out_type=jax.ShapeDtypeStruct((BATCH, N_FIELDS * EMB_DIM), jnp.float32),
        mesh=plsc.VectorSubcoreMesh(
            core_axis_name="c", subcore_axis_name="s",
            num_cores=NC, num_subcores=NS,
        ),
        scratch_types=[
            pltpu.VMEM((B_PER_W,), jnp.int32),
            pltpu.VMEM((B_PER_W, EMB_DIM), jnp.float32),
            pltpu.SemaphoreType.DMA,
        ],
        compiler_params=pltpu.CompilerParams(use_tc_tiling_on_sc=False),
    )
    return k(idx_all, *tables)


# speedup vs baseline: 1.0237x; 1.0237x over previous
import jax
import jax.numpy as jnp
from jax import lax
from jax.experimental import pallas as pl
from jax.experimental.pallas import tpu as pltpu
from jax.experimental.pallas import tpu_sc as plsc

N_FIELDS = 26
VOCAB = 100000
EMB_DIM = 32
BATCH = 16384
NC, NS = 2, 16
NW = NC * NS
B_PER_W = BATCH // NW
NBUF = 4
LOOK = 2


def _body(idx_hbm, *rest):
    tables = rest[:N_FIELDS]
    out = rest[N_FIELDS]
    idx_v = rest[N_FIELDS + 1]
    rows = rest[N_FIELDS + 2:N_FIELDS + 2 + NBUF]
    gsem = rest[N_FIELDS + 2 + NBUF:N_FIELDS + 2 + 2 * NBUF]
    wsem = rest[N_FIELDS + 2 + 2 * NBUF:N_FIELDS + 2 + 3 * NBUF]

    wid = lax.axis_index("s") * NC + lax.axis_index("c")
    base = wid * B_PER_W

    pltpu.sync_copy(idx_hbm.at[:, pl.ds(base, B_PER_W)], idx_v)

    pending_g = {}
    pending_w = {}

    def start_gather(f):
        b = f % NBUF
        pending_g[f] = pltpu.async_copy(
            tables[f].at[idx_v.at[f]], rows[b], gsem[b])

    for f in range(LOOK):
        start_gather(f)

    for f in range(N_FIELDS):
        b = f % NBUF
        pending_g.pop(f).wait()
        pending_w[f] = pltpu.async_copy(
            rows[b],
            out.at[pl.ds(base, B_PER_W), pl.ds(f * EMB_DIM, EMB_DIM)],
            wsem[b])
        g = f + LOOK
        if g < N_FIELDS:
            if g >= NBUF:
                pending_w.pop(g - NBUF).wait()
            start_gather(g)

    for f in sorted(pending_w):
        pending_w.pop(f).wait()


def kernel(idx_0, idx_1, idx_2, idx_3, idx_4, idx_5, idx_6, idx_7, idx_8, idx_9, idx_10, idx_11, idx_12, idx_13, idx_14, idx_15, idx_16, idx_17, idx_18, idx_19, idx_20, idx_21, idx_22, idx_23, idx_24, idx_25, table_0, table_1, table_2, table_3, table_4, table_5, table_6, table_7, table_8, table_9, table_10, table_11, table_12, table_13, table_14, table_15, table_16, table_17, table_18, table_19, table_20, table_21, table_22, table_23, table_24, table_25):
    idxs = [
        idx_0, idx_1, idx_2, idx_3, idx_4, idx_5, idx_6, idx_7, idx_8, idx_9,
        idx_10, idx_11, idx_12, idx_13, idx_14, idx_15, idx_16, idx_17,
        idx_18, idx_19, idx_20, idx_21, idx_22, idx_23, idx_24, idx_25,
    ]
    tables = [
        table_0, table_1, table_2, table_3, table_4, table_5, table_6,
        table_7, table_8, table_9, table_10, table_11, table_12, table_13,
        table_14, table_15, table_16, table_17, table_18, table_19, table_20,
        table_21, table_22, table_23, table_24, table_25,
    ]
    idx_all = jnp.stack(idxs).astype(jnp.int32)

    k = pl.kernel(
        _body,
        out_type=jax.ShapeDtypeStruct((BATCH, N_FIELDS * EMB_DIM), jnp.float32),
        mesh=plsc.VectorSubcoreMesh(
            core_axis_name="c", subcore_axis_name="s",
            num_cores=NC, num_subcores=NS,
        ),
        scratch_types=(
            [pltpu.VMEM((N_FIELDS, B_PER_W), jnp.int32)]
            + [pltpu.VMEM((B_PER_W, EMB_DIM), jnp.float32)] * NBUF
            + [pltpu.SemaphoreType.DMA] * (2 * NBUF)
        ),
        compiler_params=pltpu.CompilerParams(use_tc_tiling_on_sc=False),
    )
    return k(idx_all, *tables)
